# baseline (speedup 1.0000x reference)
import functools

import jax
import jax.numpy as jnp
from jax import lax
from jax.experimental import pallas as pl
from jax.experimental.pallas import tpu as pltpu
from jax.experimental.pallas import tpu_sc as plsc

_B, _T, _I = 2, 2048, 1024
_E, _K, _J = 8, 512, 512

_INFO = plsc.get_sparse_core_info()
_NC, _NS = _INFO.num_cores, _INFO.num_subcores
_NW = _NC * _NS

_CROWS = _E * _K
_RPW = _CROWS // _NW
_CHUNK = 32
_NCHUNK = _RPW // _CHUNK


def _sc_gather_batch(x_flat, ind2, b):
    boff = b * _T
    row0 = b * (_CROWS // _CHUNK)
    mesh = plsc.VectorSubcoreMesh(core_axis_name="c", subcore_axis_name="s")

    @functools.partial(
        pl.kernel,
        mesh=mesh,
        out_type=jax.ShapeDtypeStruct((_CROWS, _I), jnp.float32),
        scratch_types=[
            pltpu.VMEM((_NCHUNK, _CHUNK), jnp.int32),
            pltpu.VMEM((_CHUNK, _I), jnp.float32),
            pltpu.VMEM((_CHUNK, _I), jnp.float32),
            pltpu.SemaphoreType.DMA,
            pltpu.SemaphoreType.DMA,
            pltpu.SemaphoreType.DMA,
            pltpu.SemaphoreType.DMA,
        ],
    )
    def gather_kernel(x_hbm, ind_hbm, out_hbm, idx_v, rows0, rows1, gs0, gs1,
                      os0, os1):
        wid = lax.axis_index("s") * _NC + lax.axis_index("c")
        base = wid * _RPW

        pltpu.sync_copy(ind_hbm.at[pl.ds(row0 + wid * _NCHUNK, _NCHUNK)],
                        idx_v)
        for r in range(_NCHUNK):
            for h in range(_CHUNK // 16):
                sl = pl.ds(h * 16, 16)
                idx_v[r, sl] = idx_v[r, sl] + boff

        rows = (rows0, rows1)
        gs = (gs0, gs1)
        os = (os0, os1)

        def start_gather(c):
            return pltpu.async_copy(x_hbm.at[idx_v.at[c]], rows[c % 2],
                                    gs[c % 2])

        def start_out(c):
            return pltpu.async_copy(
                rows[c % 2], out_hbm.at[pl.ds(base + c * _CHUNK, _CHUNK)],
                os[c % 2])

        g_h = [None] * _NCHUNK
        o_h = [None] * _NCHUNK
        g_h[0] = start_gather(0)
        g_h[1] = start_gather(1)
        for c in range(_NCHUNK):
            g_h[c].wait()
            o_h[c] = start_out(c)
            if c + 2 < _NCHUNK:
                o_h[c].wait()
                g_h[c + 2] = start_gather(c + 2)
        o_h[_NCHUNK - 2].wait()
        o_h[_NCHUNK - 1].wait()

    return gather_kernel(x_flat, ind2)


def _tc_matmul_batch(xg, w, b, y_prev=None):
    def mm_kernel(*refs):
        x_ref, w_ref, o_ref = refs[-3:]
        o_ref[0, 0] = jnp.dot(x_ref[0], w_ref[0],
                              preferred_element_type=jnp.float32)

    out_shape = jax.ShapeDtypeStruct((_B, _E, _K, _J), jnp.float32)
    mm_specs = [
        pl.BlockSpec((1, _K, _I), lambda e: (e, 0, 0)),
        pl.BlockSpec((1, _I, _J), lambda e: (e, 0, 0)),
    ]
    out_spec = pl.BlockSpec((1, 1, _K, _J), lambda e: (b, e, 0, 0))
    if y_prev is None:
        return pl.pallas_call(
            mm_kernel, grid=(_E,), in_specs=mm_specs, out_specs=out_spec,
            out_shape=out_shape)(xg, w)
    return pl.pallas_call(
        mm_kernel, grid=(_E,),
        in_specs=[pl.BlockSpec(memory_space=pl.ANY)] + mm_specs,
        out_specs=out_spec, out_shape=out_shape,
        input_output_aliases={0: 0})(y_prev, xg, w)


def kernel(X, ind, W):
    x_flat = X.reshape(_B * _T, _I)
    ind2 = ind.reshape((_B * _E * _K) // _CHUNK, _CHUNK)
    xg0 = _sc_gather_batch(x_flat, ind2, 0)
    xg1 = _sc_gather_batch(x_flat, ind2, 1)
    y = _tc_matmul_batch(xg0.reshape(_E, _K, _I), W, 0)
    y = _tc_matmul_batch(xg1.reshape(_E, _K, _I), W, 1, y_prev=y)
    return y


# speedup vs baseline: 1.0481x; 1.0481x over previous
import functools

import jax
import jax.numpy as jnp
from jax import lax
from jax.experimental import pallas as pl
from jax.experimental.pallas import tpu as pltpu
from jax.experimental.pallas import tpu_sc as plsc

_B, _T, _I = 2, 2048, 1024
_E, _K, _J = 8, 512, 512

_INFO = plsc.get_sparse_core_info()
_NC, _NS = _INFO.num_cores, _INFO.num_subcores
_NW = _NC * _NS

_ES = 4
_CROWS = _B * _ES * _K
_RPW = _CROWS // _NW
_CHUNK = 32
_NCHUNK = _RPW // _CHUNK


def _sc_gather_slab(x_flat, ind2, c):
    mesh = plsc.VectorSubcoreMesh(core_axis_name="c", subcore_axis_name="s")

    @functools.partial(
        pl.kernel,
        mesh=mesh,
        out_type=jax.ShapeDtypeStruct((_CROWS, _I), jnp.float32),
        scratch_types=[
            pltpu.VMEM((_NCHUNK, _CHUNK), jnp.int32),
            pltpu.VMEM((_CHUNK, _I), jnp.float32),
            pltpu.VMEM((_CHUNK, _I), jnp.float32),
            pltpu.SemaphoreType.DMA,
            pltpu.SemaphoreType.DMA,
            pltpu.SemaphoreType.DMA,
            pltpu.SemaphoreType.DMA,
        ],
    )
    def gather_kernel(x_hbm, ind_hbm, out_hbm, idx_v, rows0, rows1, gs0, gs1,
                      os0, os1):
        wid = lax.axis_index("s") * _NC + lax.axis_index("c")
        b = wid // 16
        u = wid % 16
        boff = b * _T
        ind_blk = ((b * _E + _ES * c + u // 4) * _K + (u % 4) * _RPW) // _RPW
        base = wid * _RPW

        pltpu.sync_copy(ind_hbm.at[ind_blk], idx_v)
        for r in range(_NCHUNK):
            for h in range(_CHUNK // 16):
                sl = pl.ds(h * 16, 16)
                idx_v[r, sl] = idx_v[r, sl] + boff

        rows = (rows0, rows1)
        gs = (gs0, gs1)
        os = (os0, os1)

        def start_gather(s):
            return pltpu.async_copy(x_hbm.at[idx_v.at[s]], rows[s % 2],
                                    gs[s % 2])

        def start_out(s):
            return pltpu.async_copy(
                rows[s % 2], out_hbm.at[pl.ds(base + s * _CHUNK, _CHUNK)],
                os[s % 2])

        g_h = [None] * _NCHUNK
        o_h = [None] * _NCHUNK
        g_h[0] = start_gather(0)
        g_h[1] = start_gather(1)
        for s in range(_NCHUNK):
            g_h[s].wait()
            o_h[s] = start_out(s)
            if s + 2 < _NCHUNK:
                o_h[s].wait()
                g_h[s + 2] = start_gather(s + 2)
        o_h[_NCHUNK - 2].wait()
        o_h[_NCHUNK - 1].wait()

    return gather_kernel(x_flat, ind2)


def _tc_matmul_slab(xg, w, c, y_prev=None):
    def mm_kernel(*refs):
        x_ref, w_ref, o_ref = refs[-3:]
        o_ref[0, 0] = jnp.dot(x_ref[0, 0], w_ref[0],
                              preferred_element_type=jnp.float32)

    out_shape = jax.ShapeDtypeStruct((_B, _E, _K, _J), jnp.float32)
    mm_specs = [
        pl.BlockSpec((1, 1, _K, _I), lambda e, b: (b, e, 0, 0)),
        pl.BlockSpec((1, _I, _J), lambda e, b: (_ES * c + e, 0, 0)),
    ]
    out_spec = pl.BlockSpec((1, 1, _K, _J),
                            lambda e, b: (b, _ES * c + e, 0, 0))
    if y_prev is None:
        return pl.pallas_call(
            mm_kernel, grid=(_ES, _B), in_specs=mm_specs, out_specs=out_spec,
            out_shape=out_shape)(xg, w)
    return pl.pallas_call(
        mm_kernel, grid=(_ES, _B),
        in_specs=[pl.BlockSpec(memory_space=pl.ANY)] + mm_specs,
        out_specs=out_spec, out_shape=out_shape,
        input_output_aliases={0: 0})(y_prev, xg, w)


def kernel(X, ind, W):
    x_flat = X.reshape(_B * _T, _I)
    ind2 = ind.reshape((_B * _E * _K) // _RPW, _NCHUNK, _CHUNK)
    xg0 = _sc_gather_slab(x_flat, ind2, 0)
    xg1 = _sc_gather_slab(x_flat, ind2, 1)
    y = _tc_matmul_slab(xg0.reshape(_B, _ES, _K, _I), W, 0)
    y = _tc_matmul_slab(xg1.reshape(_B, _ES, _K, _I), W, 1, y_prev=y)
    return y


# speedup vs baseline: 1.0941x; 1.0440x over previous
import functools

import jax
import jax.numpy as jnp
from jax import lax
from jax.experimental import pallas as pl
from jax.experimental.pallas import tpu as pltpu
from jax.experimental.pallas import tpu_sc as plsc

_B, _T, _I = 2, 2048, 1024
_E, _K, _J = 8, 512, 512

_INFO = plsc.get_sparse_core_info()
_NC, _NS = _INFO.num_cores, _INFO.num_subcores
_NW = _NC * _NS

_ES = 4
_CROWS = _B * _ES * _K
_RPW = _CROWS // _NW
_CHUNK = 32
_NCHUNK = _RPW // _CHUNK


def _sc_gather_slab(x_flat, ind2, c):
    mesh = plsc.VectorSubcoreMesh(core_axis_name="c", subcore_axis_name="s")

    @functools.partial(
        pl.kernel,
        mesh=mesh,
        out_type=jax.ShapeDtypeStruct((_CROWS, _I), jnp.float32),
        scratch_types=[
            pltpu.VMEM((_NCHUNK, _CHUNK), jnp.int32),
            pltpu.VMEM((_CHUNK, _I), jnp.float32),
            pltpu.VMEM((_CHUNK, _I), jnp.float32),
            pltpu.SemaphoreType.DMA,
            pltpu.SemaphoreType.DMA,
            pltpu.SemaphoreType.DMA,
            pltpu.SemaphoreType.DMA,
        ],
    )
    def gather_kernel(x_hbm, ind_hbm, out_hbm, idx_v, rows0, rows1, gs0, gs1,
                      os0, os1):
        wid = lax.axis_index("s") * _NC + lax.axis_index("c")
        b = wid // 16
        u = wid % 16
        boff = b * _T
        ind_blk = ((b * _E + _ES * c + u // 4) * _K + (u % 4) * _RPW) // _RPW
        base = wid * _RPW

        pltpu.sync_copy(ind_hbm.at[ind_blk], idx_v)
        for r in range(_NCHUNK):
            for h in range(_CHUNK // 16):
                sl = pl.ds(h * 16, 16)
                idx_v[r, sl] = idx_v[r, sl] + boff

        rows = (rows0, rows1)
        gs = (gs0, gs1)
        os = (os0, os1)

        def start_gather(s):
            return pltpu.async_copy(x_hbm.at[idx_v.at[s]], rows[s % 2],
                                    gs[s % 2])

        def start_out(s):
            return pltpu.async_copy(
                rows[s % 2], out_hbm.at[pl.ds(base + s * _CHUNK, _CHUNK)],
                os[s % 2])

        g_h = [None] * _NCHUNK
        o_h = [None] * _NCHUNK
        g_h[0] = start_gather(0)
        g_h[1] = start_gather(1)
        for s in range(_NCHUNK):
            g_h[s].wait()
            o_h[s] = start_out(s)
            if s + 2 < _NCHUNK:
                o_h[s].wait()
                g_h[s + 2] = start_gather(s + 2)
        o_h[_NCHUNK - 2].wait()
        o_h[_NCHUNK - 1].wait()

    return gather_kernel(x_flat, ind2)


def _tc_matmul_slab(xg, w, c, y_prev=None):
    def mm_kernel(*refs):
        x_ref, w_ref, o_ref = refs[-3:]
        for bi in range(_B):
            o_ref[bi, 0] = jnp.dot(x_ref[bi, 0], w_ref[0],
                                   preferred_element_type=jnp.float32)

    out_shape = jax.ShapeDtypeStruct((_B, _E, _K, _J), jnp.float32)
    mm_specs = [
        pl.BlockSpec((_B, 1, _K, _I), lambda e: (0, e, 0, 0)),
        pl.BlockSpec((1, _I, _J), lambda e: (_ES * c + e, 0, 0)),
    ]
    out_spec = pl.BlockSpec((_B, 1, _K, _J),
                            lambda e: (0, _ES * c + e, 0, 0))
    if y_prev is None:
        return pl.pallas_call(
            mm_kernel, grid=(_ES,), in_specs=mm_specs, out_specs=out_spec,
            out_shape=out_shape)(xg, w)
    return pl.pallas_call(
        mm_kernel, grid=(_ES,),
        in_specs=[pl.BlockSpec(memory_space=pl.ANY)] + mm_specs,
        out_specs=out_spec, out_shape=out_shape,
        input_output_aliases={0: 0})(y_prev, xg, w)


def kernel(X, ind, W):
    x_flat = X.reshape(_B * _T, _I)
    ind2 = ind.reshape((_B * _E * _K) // _RPW, _NCHUNK, _CHUNK)
    xg0 = _sc_gather_slab(x_flat, ind2, 0)
    xg1 = _sc_gather_slab(x_flat, ind2, 1)
    y = _tc_matmul_slab(xg0.reshape(_B, _ES, _K, _I), W, 0)
    y = _tc_matmul_slab(xg1.reshape(_B, _ES, _K, _I), W, 1, y_prev=y)
    return y
